# baseline (speedup 1.0000x reference)
import jax
import jax.numpy as jnp
from jax.experimental import pallas as pl
from jax.experimental.pallas import tpu as pltpu

BR = 512
RS = 16
S_STATIC = 8192


def _fused_body(x_ref, mask_ref, o_ref, carry_ref):
    i = pl.program_id(0)
    c0 = carry_ref[...]
    c0 = jnp.where((i * BR) % S_STATIC == 0, jnp.zeros_like(c0), c0)

    def strip(s, prev):
        base = s * RS
        rows = x_ref[pl.ds(base, RS)]
        ext = jnp.concatenate([prev[5:], rows], axis=0)
        m2 = jnp.minimum(ext[1:], ext[:-1])
        m = jnp.minimum(m2[2:], m2[:RS])
        sig = jax.nn.sigmoid(rows)
        w = mask_ref[pl.ds(base, RS)] * sig
        o_ref[pl.ds(base, RS)] = rows + w * (m - rows)
        return rows[RS - 8:]

    last = jax.lax.fori_loop(0, BR // RS, strip, c0)
    carry_ref[...] = last


def kernel(_x, rand_index):
    B, S, H = _x.shape
    assert S == S_STATIC and S % BR == 0
    T = B * S
    xf = _x.reshape(T, H)
    mask = jnp.zeros((T, 1), jnp.float32).at[rand_index].set(1.0)

    out = pl.pallas_call(
        _fused_body,
        grid=(T // BR,),
        in_specs=[
            pl.BlockSpec((BR, H), lambda i: (i, 0)),
            pl.BlockSpec((BR, 1), lambda i: (i, 0)),
        ],
        out_specs=pl.BlockSpec((BR, H), lambda i: (i, 0)),
        out_shape=jax.ShapeDtypeStruct((T, H), jnp.float32),
        scratch_shapes=[pltpu.VMEM((8, H), jnp.float32)],
    )(xf, mask)
    return out.reshape(B, S, H)


# speedup vs baseline: 1.1367x; 1.1367x over previous
import jax
import jax.numpy as jnp
from jax.experimental import pallas as pl
from jax.experimental.pallas import tpu as pltpu

BR = 512
RS = 16
S_STATIC = 8192


def _fused_body(x_ref, mask_ref, o_ref, carry_ref):
    i = pl.program_id(0)
    c0 = carry_ref[...]
    c0 = jnp.where((i * BR) % S_STATIC == 0, jnp.zeros_like(c0), c0)

    prev = c0
    for s in range(BR // RS):
        base = s * RS
        rows = x_ref[base:base + RS]
        ext = jnp.concatenate([prev[5:], rows], axis=0)
        m2 = jnp.minimum(ext[1:], ext[:-1])
        m = jnp.minimum(m2[2:], m2[:RS])
        sig = jax.nn.sigmoid(rows)
        w = mask_ref[base:base + RS] * sig
        o_ref[base:base + RS] = rows + w * (m - rows)
        prev = rows[RS - 8:]

    carry_ref[...] = prev


def kernel(_x, rand_index):
    B, S, H = _x.shape
    assert S == S_STATIC and S % BR == 0
    T = B * S
    xf = _x.reshape(T, H)
    mask = jnp.zeros((T, 1), jnp.float32).at[rand_index].set(1.0)

    out = pl.pallas_call(
        _fused_body,
        grid=(T // BR,),
        in_specs=[
            pl.BlockSpec((BR, H), lambda i: (i, 0)),
            pl.BlockSpec((BR, 1), lambda i: (i, 0)),
        ],
        out_specs=pl.BlockSpec((BR, H), lambda i: (i, 0)),
        out_shape=jax.ShapeDtypeStruct((T, H), jnp.float32),
        scratch_shapes=[pltpu.VMEM((8, H), jnp.float32)],
    )(xf, mask)
    return out.reshape(B, S, H)


# speedup vs baseline: 1.2138x; 1.0678x over previous
import jax
import jax.numpy as jnp
from jax.experimental import pallas as pl
from jax.experimental.pallas import tpu as pltpu

BR = 1024
RS = 16
S_STATIC = 8192


def _fused_body(x_ref, mask_ref, o_ref, carry_ref):
    i = pl.program_id(0)
    c0 = carry_ref[...]
    c0 = jnp.where((i * BR) % S_STATIC == 0, jnp.zeros_like(c0), c0)

    prev = c0
    for s in range(BR // RS):
        base = s * RS
        rows = x_ref[base:base + RS]
        ext = jnp.concatenate([prev[5:], rows], axis=0)
        m2 = jnp.minimum(ext[1:], ext[:-1])
        m = jnp.minimum(m2[2:], m2[:RS])
        sig = jax.nn.sigmoid(rows)
        w = mask_ref[base:base + RS] * sig
        o_ref[base:base + RS] = rows + w * (m - rows)
        prev = rows[RS - 8:]

    carry_ref[...] = prev


def kernel(_x, rand_index):
    B, S, H = _x.shape
    assert S == S_STATIC and S % BR == 0
    T = B * S
    xf = _x.reshape(T, H)
    mask = jnp.zeros((T, 1), jnp.float32).at[rand_index].set(1.0)

    out = pl.pallas_call(
        _fused_body,
        grid=(T // BR,),
        in_specs=[
            pl.BlockSpec((BR, H), lambda i: (i, 0)),
            pl.BlockSpec((BR, 1), lambda i: (i, 0)),
        ],
        out_specs=pl.BlockSpec((BR, H), lambda i: (i, 0)),
        out_shape=jax.ShapeDtypeStruct((T, H), jnp.float32),
        scratch_shapes=[pltpu.VMEM((8, H), jnp.float32)],
    )(xf, mask)
    return out.reshape(B, S, H)


# speedup vs baseline: 1.2354x; 1.0179x over previous
import jax
import jax.numpy as jnp
from jax.experimental import pallas as pl
from jax.experimental.pallas import tpu as pltpu

BR = 2048
RS = 16
S_STATIC = 8192


def _fused_body(x_ref, mask_ref, o_ref, carry_ref):
    i = pl.program_id(0)
    c0 = carry_ref[...]
    c0 = jnp.where((i * BR) % S_STATIC == 0, jnp.zeros_like(c0), c0)

    prev = c0
    for s in range(BR // RS):
        base = s * RS
        rows = x_ref[base:base + RS]
        ext = jnp.concatenate([prev[5:], rows], axis=0)
        m2 = jnp.minimum(ext[1:], ext[:-1])
        m = jnp.minimum(m2[2:], m2[:RS])
        sig = jax.nn.sigmoid(rows)
        w = mask_ref[base:base + RS] * sig
        o_ref[base:base + RS] = rows + w * (m - rows)
        prev = rows[RS - 8:]

    carry_ref[...] = prev


def kernel(_x, rand_index):
    B, S, H = _x.shape
    assert S == S_STATIC and S % BR == 0
    T = B * S
    xf = _x.reshape(T, H)
    mask = jnp.zeros((T, 1), jnp.float32).at[rand_index].set(1.0)

    out = pl.pallas_call(
        _fused_body,
        grid=(T // BR,),
        in_specs=[
            pl.BlockSpec((BR, H), lambda i: (i, 0)),
            pl.BlockSpec((BR, 1), lambda i: (i, 0)),
        ],
        out_specs=pl.BlockSpec((BR, H), lambda i: (i, 0)),
        out_shape=jax.ShapeDtypeStruct((T, H), jnp.float32),
        scratch_shapes=[pltpu.VMEM((8, H), jnp.float32)],
    )(xf, mask)
    return out.reshape(B, S, H)
